# Initial kernel scaffold; baseline (speedup 1.0000x reference)
import functools

import jax
import jax.numpy as jnp
from jax.experimental import pallas as pl
from jax.experimental.pallas import tpu as pltpu

B, S, D = 1, 2048, 768
H, KR = 12, 64
E, TOPK, DFF = 8, 2, 1536

SB = 256
TB = 512
NB = S // SB
NQ = S // TB


def _ln(h, g, b):
    mu = jnp.mean(h, axis=-1, keepdims=True)
    var = jnp.mean((h - mu) ** 2, axis=-1, keepdims=True)
    return (h - mu) * jax.lax.rsqrt(var + 1e-5) * g + b


def _elu1(x):
    return jnp.where(x > 0, x + 1.0, jnp.exp(x))


def _attn_kernel(x_ref, wq_ref, wk_ref, wv_ref, wo_ref, g_ref, b_ref,
                 out_ref, kv_ref, ksum_ref):
    p = pl.program_id(0)
    bb = pl.program_id(1)

    @pl.when((p == 0) & (bb == 0))
    def _init():
        kv_ref[...] = jnp.zeros_like(kv_ref)
        ksum_ref[...] = jnp.zeros_like(ksum_ref)

    x_blk = x_ref[...]
    h = _ln(x_blk, g_ref[...], b_ref[...])

    @pl.when(p == 0)
    def _acc():
        k = h @ wk_ref[...]
        v = h @ wv_ref[...]
        pk = _elu1(k)
        kv_ref[...] += jax.lax.dot_general(
            pk, v, dimension_numbers=(((0,), (0,)), ((), ())))
        ksum_ref[0:1, :] += jnp.sum(pk, axis=0, keepdims=True)

    @pl.when(p == 1)
    def _out():
        q = h @ wq_ref[...]
        pq = _elu1(q)
        ksum = ksum_ref[0:1, :]
        cols = []
        for hh in range(H):
            sl = slice(hh * KR, (hh + 1) * KR)
            num_h = pq[:, sl] @ kv_ref[sl, sl]
            den_h = jnp.sum(pq[:, sl] * ksum[:, sl], axis=1, keepdims=True)
            cols.append(num_h / (den_h + 1e-6))
        attn_v = jnp.concatenate(cols, axis=1)
        out_ref[...] = x_blk + attn_v @ wo_ref[...]


def _attention(x2d, Wq, Wk, Wv, Wo, g, b):
    return pl.pallas_call(
        _attn_kernel,
        grid=(2, NB),
        in_specs=[
            pl.BlockSpec((SB, D), lambda p, bb: (bb, 0)),
            pl.BlockSpec((D, D), lambda p, bb: (0, 0)),
            pl.BlockSpec((D, D), lambda p, bb: (0, 0)),
            pl.BlockSpec((D, D), lambda p, bb: (0, 0)),
            pl.BlockSpec((D, D), lambda p, bb: (0, 0)),
            pl.BlockSpec((1, D), lambda p, bb: (0, 0)),
            pl.BlockSpec((1, D), lambda p, bb: (0, 0)),
        ],
        out_specs=pl.BlockSpec((SB, D), lambda p, bb: (bb, 0)),
        out_shape=jax.ShapeDtypeStruct((S, D), jnp.float32),
        scratch_shapes=[
            pltpu.VMEM((D, D), jnp.float32),
            pltpu.VMEM((8, D), jnp.float32),
        ],
        compiler_params=pltpu.CompilerParams(
            dimension_semantics=("arbitrary", "arbitrary")),
    )(x2d, Wq, Wk, Wv, Wo, g.reshape(1, D), b.reshape(1, D))


def _router_kernel(x1_ref, g_ref, b_ref, wr_ref, t_ref, c_ref):
    t = _ln(x1_ref[...], g_ref[...], b_ref[...])
    t_ref[...] = t
    logits = t @ wr_ref[...]
    m = jnp.max(logits, axis=1, keepdims=True)
    ex = jnp.exp(logits - m)
    p = ex / jnp.sum(ex, axis=1, keepdims=True)
    iota = jax.lax.broadcasted_iota(jnp.int32, (S, E), 1)
    m0 = jnp.max(p, axis=1, keepdims=True)
    i0 = jnp.min(jnp.where(p == m0, iota, E), axis=1, keepdims=True)
    sel0 = iota == i0
    p1 = jnp.where(sel0, -1.0, p)
    m1 = jnp.max(p1, axis=1, keepdims=True)
    i1 = jnp.min(jnp.where(p1 == m1, iota, E), axis=1, keepdims=True)
    sel1 = iota == i1
    denom = m0 + m1
    c_ref[...] = jnp.where(sel0, m0 / denom, 0.0) + jnp.where(sel1, m1 / denom, 0.0)


def _router(x1, g, b, Wr):
    return pl.pallas_call(
        _router_kernel,
        grid=(1,),
        in_specs=[
            pl.BlockSpec((S, D), lambda i: (0, 0)),
            pl.BlockSpec((1, D), lambda i: (0, 0)),
            pl.BlockSpec((1, D), lambda i: (0, 0)),
            pl.BlockSpec((D, E), lambda i: (0, 0)),
        ],
        out_specs=[
            pl.BlockSpec((S, D), lambda i: (0, 0)),
            pl.BlockSpec((S, E), lambda i: (0, 0)),
        ],
        out_shape=[
            jax.ShapeDtypeStruct((S, D), jnp.float32),
            jax.ShapeDtypeStruct((S, E), jnp.float32),
        ],
    )(x1, g.reshape(1, D), b.reshape(1, D), Wr)


def _moe_kernel(t_ref, x1_ref, c_ref, w1_ref, b1_ref, w2_ref, b2_ref, out_ref):
    e = pl.program_id(0)
    q = pl.program_id(1)
    t_blk = t_ref[...]
    h1 = jax.nn.gelu(t_blk @ w1_ref[0] + b1_ref[...])
    y = h1 @ w2_ref[0] + b2_ref[...]
    iota = jax.lax.broadcasted_iota(jnp.int32, (1, E), 1)
    ce = jnp.sum(jnp.where(iota == e, c_ref[...], 0.0), axis=1, keepdims=True)
    contrib = ce * y
    ds = pl.ds(q * TB, TB)

    @pl.when(e == 0)
    def _first():
        out_ref[ds, :] = x1_ref[...] + contrib

    @pl.when(e > 0)
    def _rest():
        out_ref[ds, :] += contrib


def _moe_dense(t, x1, C, W1, b1, W2, b2):
    return pl.pallas_call(
        _moe_kernel,
        grid=(E, NQ),
        in_specs=[
            pl.BlockSpec((TB, D), lambda e, q: (q, 0)),
            pl.BlockSpec((TB, D), lambda e, q: (q, 0)),
            pl.BlockSpec((TB, E), lambda e, q: (q, 0)),
            pl.BlockSpec((1, D, DFF), lambda e, q: (e, 0, 0)),
            pl.BlockSpec((1, DFF), lambda e, q: (e, 0)),
            pl.BlockSpec((1, DFF, D), lambda e, q: (e, 0, 0)),
            pl.BlockSpec((1, D), lambda e, q: (e, 0)),
        ],
        out_specs=pl.BlockSpec((S, D), lambda e, q: (0, 0)),
        out_shape=jax.ShapeDtypeStruct((S, D), jnp.float32),
        compiler_params=pltpu.CompilerParams(
            dimension_semantics=("arbitrary", "arbitrary")),
    )(t, x1, C, W1, b1, W2, b2)


@jax.jit
def kernel(x, Wq, Wk, Wv, Wo, ln1_g, ln1_b, ln2_g, ln2_b, Wr, W1, b1, W2, b2):
    x2d = x.reshape(S, D)
    x1 = _attention(x2d, Wq, Wk, Wv, Wo, ln1_g, ln1_b)
    t, C = _router(x1, ln2_g, ln2_b, Wr)
    out = _moe_dense(t, x1, C, W1, b1, W2, b2)
    return out.reshape(B, S, D)


# speedup vs baseline: 1.3626x; 1.3626x over previous
import functools

import jax
import jax.numpy as jnp
from jax.experimental import pallas as pl
from jax.experimental.pallas import tpu as pltpu

B, S, D = 1, 2048, 768
H, KR = 12, 64
E, TOPK, DFF = 8, 2, 1536

SB = 256
TB = 512
NB = S // SB
NQ = S // TB


def _ln(h, g, b):
    mu = jnp.mean(h, axis=-1, keepdims=True)
    var = jnp.mean((h - mu) ** 2, axis=-1, keepdims=True)
    return (h - mu) * jax.lax.rsqrt(var + 1e-5) * g + b


def _elu1(x):
    return jnp.where(x > 0, x + 1.0, jnp.exp(x))


def _attn_kv_kernel(x_ref, wk_ref, wv_ref, g_ref, b_ref,
                    kv_ref, ksum_ref, kv_acc, ks_acc):
    bb = pl.program_id(0)

    @pl.when(bb == 0)
    def _init():
        kv_acc[...] = jnp.zeros_like(kv_acc)
        ks_acc[...] = jnp.zeros_like(ks_acc)

    h = _ln(x_ref[...], g_ref[...], b_ref[...])
    k = h @ wk_ref[...]
    v = h @ wv_ref[...]
    pk = _elu1(k)
    kv_acc[...] += jax.lax.dot_general(
        pk, v, dimension_numbers=(((0,), (0,)), ((), ())))
    ks_acc[0:1, :] += jnp.sum(pk, axis=0, keepdims=True)

    @pl.when(bb == NB - 1)
    def _fin():
        kv_ref[...] = kv_acc[...]
        ksum_ref[...] = ks_acc[...]


def _attn_out_kernel(x_ref, wq_ref, wo_ref, g_ref, b_ref, kv_ref, ksum_ref,
                     out_ref):
    x_blk = x_ref[...]
    h = _ln(x_blk, g_ref[...], b_ref[...])
    q = h @ wq_ref[...]
    pq = _elu1(q)
    ksum = ksum_ref[0:1, :]
    cols = []
    for hh in range(H):
        sl = slice(hh * KR, (hh + 1) * KR)
        num_h = pq[:, sl] @ kv_ref[sl, sl]
        den_h = jnp.sum(pq[:, sl] * ksum[:, sl], axis=1, keepdims=True)
        cols.append(num_h / (den_h + 1e-6))
    attn_v = jnp.concatenate(cols, axis=1)
    out_ref[...] = x_blk + attn_v @ wo_ref[...]


def _attention(x2d, Wq, Wk, Wv, Wo, g, b):
    g2 = g.reshape(1, D)
    b2 = b.reshape(1, D)
    kv, ksum = pl.pallas_call(
        _attn_kv_kernel,
        grid=(NB,),
        in_specs=[
            pl.BlockSpec((SB, D), lambda bb: (bb, 0)),
            pl.BlockSpec((D, D), lambda bb: (0, 0)),
            pl.BlockSpec((D, D), lambda bb: (0, 0)),
            pl.BlockSpec((1, D), lambda bb: (0, 0)),
            pl.BlockSpec((1, D), lambda bb: (0, 0)),
        ],
        out_specs=[
            pl.BlockSpec((D, D), lambda bb: (0, 0)),
            pl.BlockSpec((8, D), lambda bb: (0, 0)),
        ],
        out_shape=[
            jax.ShapeDtypeStruct((D, D), jnp.float32),
            jax.ShapeDtypeStruct((8, D), jnp.float32),
        ],
        scratch_shapes=[
            pltpu.VMEM((D, D), jnp.float32),
            pltpu.VMEM((8, D), jnp.float32),
        ],
        compiler_params=pltpu.CompilerParams(
            dimension_semantics=("arbitrary",)),
    )(x2d, Wk, Wv, g2, b2)
    return pl.pallas_call(
        _attn_out_kernel,
        grid=(NB,),
        in_specs=[
            pl.BlockSpec((SB, D), lambda bb: (bb, 0)),
            pl.BlockSpec((D, D), lambda bb: (0, 0)),
            pl.BlockSpec((D, D), lambda bb: (0, 0)),
            pl.BlockSpec((1, D), lambda bb: (0, 0)),
            pl.BlockSpec((1, D), lambda bb: (0, 0)),
            pl.BlockSpec((D, D), lambda bb: (0, 0)),
            pl.BlockSpec((8, D), lambda bb: (0, 0)),
        ],
        out_specs=pl.BlockSpec((SB, D), lambda bb: (bb, 0)),
        out_shape=jax.ShapeDtypeStruct((S, D), jnp.float32),
        compiler_params=pltpu.CompilerParams(
            dimension_semantics=("arbitrary",)),
    )(x2d, Wq, Wo, g2, b2, kv, ksum)


def _router_kernel(x1_ref, g_ref, b_ref, wr_ref, t_ref, c_ref):
    t = _ln(x1_ref[...], g_ref[...], b_ref[...])
    t_ref[...] = t
    logits = t @ wr_ref[...]
    m = jnp.max(logits, axis=1, keepdims=True)
    ex = jnp.exp(logits - m)
    p = ex / jnp.sum(ex, axis=1, keepdims=True)
    iota = jax.lax.broadcasted_iota(jnp.int32, (S, E), 1)
    m0 = jnp.max(p, axis=1, keepdims=True)
    i0 = jnp.min(jnp.where(p == m0, iota, E), axis=1, keepdims=True)
    sel0 = iota == i0
    p1 = jnp.where(sel0, -1.0, p)
    m1 = jnp.max(p1, axis=1, keepdims=True)
    i1 = jnp.min(jnp.where(p1 == m1, iota, E), axis=1, keepdims=True)
    sel1 = iota == i1
    denom = m0 + m1
    c_ref[...] = jnp.where(sel0, m0 / denom, 0.0) + jnp.where(sel1, m1 / denom, 0.0)


def _router(x1, g, b, Wr):
    return pl.pallas_call(
        _router_kernel,
        grid=(1,),
        in_specs=[
            pl.BlockSpec((S, D), lambda i: (0, 0)),
            pl.BlockSpec((1, D), lambda i: (0, 0)),
            pl.BlockSpec((1, D), lambda i: (0, 0)),
            pl.BlockSpec((D, E), lambda i: (0, 0)),
        ],
        out_specs=[
            pl.BlockSpec((S, D), lambda i: (0, 0)),
            pl.BlockSpec((S, E), lambda i: (0, 0)),
        ],
        out_shape=[
            jax.ShapeDtypeStruct((S, D), jnp.float32),
            jax.ShapeDtypeStruct((S, E), jnp.float32),
        ],
    )(x1, g.reshape(1, D), b.reshape(1, D), Wr)


def _moe_kernel(t_ref, x1_ref, c_ref, w1_ref, b1_ref, w2_ref, b2_ref, out_ref):
    e = pl.program_id(0)
    q = pl.program_id(1)
    t_blk = t_ref[...]
    h1 = jax.nn.gelu(t_blk @ w1_ref[0] + b1_ref[0])
    y = h1 @ w2_ref[0] + b2_ref[0]
    iota = jax.lax.broadcasted_iota(jnp.int32, (1, E), 1)
    ce = jnp.sum(jnp.where(iota == e, c_ref[...], 0.0), axis=1, keepdims=True)
    contrib = ce * y
    ds = pl.ds(q * TB, TB)

    @pl.when(e == 0)
    def _first():
        out_ref[ds, :] = x1_ref[...] + contrib

    @pl.when(e > 0)
    def _rest():
        out_ref[ds, :] += contrib


def _moe_dense(t, x1, C, W1, b1, W2, b2):
    return pl.pallas_call(
        _moe_kernel,
        grid=(E, NQ),
        in_specs=[
            pl.BlockSpec((TB, D), lambda e, q: (q, 0)),
            pl.BlockSpec((TB, D), lambda e, q: (q, 0)),
            pl.BlockSpec((TB, E), lambda e, q: (q, 0)),
            pl.BlockSpec((1, D, DFF), lambda e, q: (e, 0, 0)),
            pl.BlockSpec((1, 1, DFF), lambda e, q: (e, 0, 0)),
            pl.BlockSpec((1, DFF, D), lambda e, q: (e, 0, 0)),
            pl.BlockSpec((1, 1, D), lambda e, q: (e, 0, 0)),
        ],
        out_specs=pl.BlockSpec((S, D), lambda e, q: (0, 0)),
        out_shape=jax.ShapeDtypeStruct((S, D), jnp.float32),
        compiler_params=pltpu.CompilerParams(
            dimension_semantics=("arbitrary", "arbitrary")),
    )(t, x1, C, W1, b1.reshape(E, 1, DFF), W2, b2.reshape(E, 1, D))


@jax.jit
def kernel(x, Wq, Wk, Wv, Wo, ln1_g, ln1_b, ln2_g, ln2_b, Wr, W1, b1, W2, b2):
    x2d = x.reshape(S, D)
    x1 = _attention(x2d, Wq, Wk, Wv, Wo, ln1_g, ln1_b)
    t, C = _router(x1, ln2_g, ln2_b, Wr)
    out = _moe_dense(t, x1, C, W1, b1, W2, b2)
    return out.reshape(B, S, D)
